# baseline (speedup 1.0000x reference)
import functools

import jax
import jax.numpy as jnp
from jax import lax
from jax.experimental import pallas as pl
from jax.experimental.pallas import tpu as pltpu
from jax.experimental.pallas import tpu_sc as plsc

VOCAB = 1000000
EMBED = 64
BATCH = 16384
PAIRW = 2 * EMBED

VBLK = 4096
VBITS = 12
NVB = (VOCAB + VBLK - 1) // VBLK
NPB = (NVB + 1) // 2
PROWS = NPB * VBLK

NC = 2
NS = 16
L = 16
NW = NC * NS
BPW = BATCH // NW
NCHUNK = 4
CH = BPW // NCHUNK


def _repack_body(wlo_ref, whi_ref, clo_ref, chi_ref, wout_ref, cout_ref):
    wout_ref[...] = jnp.concatenate([wlo_ref[...].T, whi_ref[...].T], axis=1)
    cout_ref[...] = jnp.concatenate([clo_ref[...].T, chi_ref[...].T], axis=1)


def _hi_block(i):
    return (0, jnp.minimum(2 * i + 1, NVB - 1))


_repack = pl.pallas_call(
    _repack_body,
    grid=(NPB,),
    in_specs=[
        pl.BlockSpec((EMBED, VBLK), lambda i: (0, 2 * i)),
        pl.BlockSpec((EMBED, VBLK), _hi_block),
        pl.BlockSpec((EMBED, VBLK), lambda i: (0, 2 * i)),
        pl.BlockSpec((EMBED, VBLK), _hi_block),
    ],
    out_specs=[
        pl.BlockSpec((VBLK, PAIRW), lambda i: (i, 0)),
        pl.BlockSpec((VBLK, PAIRW), lambda i: (i, 0)),
    ],
    out_shape=[
        jax.ShapeDtypeStruct((PROWS, PAIRW), jnp.float32),
        jax.ShapeDtypeStruct((PROWS, PAIRW), jnp.float32),
    ],
)


_mesh = plsc.VectorSubcoreMesh(core_axis_name="c", subcore_axis_name="s")


@functools.partial(
    pl.kernel,
    out_type=jax.ShapeDtypeStruct((BATCH,), jnp.float32),
    mesh=_mesh,
    compiler_params=pltpu.CompilerParams(needs_layout_passes=False,
                                         use_tc_tiling_on_sc=True),
    scratch_types=[
        pltpu.VMEM((NCHUNK, CH), jnp.int32),
        pltpu.VMEM((NCHUNK, CH), jnp.int32),
        pltpu.VMEM((NCHUNK, CH), jnp.int32),
        pltpu.VMEM((NCHUNK, CH), jnp.int32),
        pltpu.VMEM((NCHUNK, CH), jnp.int32),
        pltpu.VMEM((NCHUNK, CH), jnp.int32),
        pltpu.VMEM((CH, PAIRW), jnp.float32),
        pltpu.VMEM((CH, PAIRW), jnp.float32),
        pltpu.VMEM((CH, PAIRW), jnp.float32),
        pltpu.VMEM((CH, PAIRW), jnp.float32),
        pltpu.VMEM((BPW,), jnp.float32),
        pltpu.VMEM((BPW,), jnp.float32),
        pltpu.VMEM((BPW,), jnp.float32),
        pltpu.SemaphoreType.DMA,
        pltpu.SemaphoreType.DMA,
    ],
)
def _sc_kernel(widx_hbm, cidx_hbm, wtab_hbm, ctab_hbm, wb_hbm, cb_hbm,
               out_hbm, widx_v, cidx_v, wp_v, cp_v, wo_v, co_v,
               wrows0, wrows1, crows0, crows1, wb_v, cb_v, out_v, sem, bsem):
    wid = lax.axis_index("s") * NC + lax.axis_index("c")
    base = wid * BPW
    wbufs = (wrows0, wrows1)
    cbufs = (crows0, crows1)

    pltpu.sync_copy(widx_hbm.at[wid], widx_v)
    pltpu.sync_copy(cidx_hbm.at[wid], cidx_v)

    for j in range(NCHUNK):
        for t in range(CH // L):
            sl = pl.ds(t * L, L)
            for iv, pv, ov in ((widx_v, wp_v, wo_v), (cidx_v, cp_v, co_v)):
                v = iv[j, sl]
                k = lax.shift_right_logical(v, VBITS)
                pv[j, sl] = lax.shift_left(lax.shift_right_logical(k, 1), VBITS) + (v & (VBLK - 1))
                ov[j, sl] = (k & 1) * EMBED

    def fire(j):
        return (pltpu.async_copy(wtab_hbm.at[wp_v.at[j]], wbufs[j % 2], sem),
                pltpu.async_copy(ctab_hbm.at[cp_v.at[j]], cbufs[j % 2], sem))

    inflight = fire(0)

    bias_copies = []
    for j in range(NCHUNK):
        sl = pl.ds(j * CH, CH)
        bias_copies.append(pltpu.async_copy(wb_hbm.at[widx_v.at[j]], wb_v.at[sl], bsem))
        bias_copies.append(pltpu.async_copy(cb_hbm.at[cidx_v.at[j]], cb_v.at[sl], bsem))

    lane = lax.iota(jnp.int32, L)

    for j in range(NCHUNK):
        for c in inflight:
            c.wait()
        if j + 1 < NCHUNK:
            inflight = fire(j + 1)
        if j == 0:
            for c in bias_copies:
                c.wait()
        wrows, crows = wbufs[j % 2], cbufs[j % 2]

        def group(g, carry, j=j, wrows=wrows, crows=crows):
            b16 = g * L + lane
            osl = pl.ds(j * CH + g * L, L)
            woff = wo_v[j, pl.ds(g * L, L)]
            coff = co_v[j, pl.ds(g * L, L)]
            accs = [jnp.zeros((L,), jnp.float32) for _ in range(4)]
            for d in range(EMBED):
                wv = plsc.load_gather(wrows, [b16, woff + d])
                cv = plsc.load_gather(crows, [b16, coff + d])
                accs[d % 4] = accs[d % 4] + wv * cv
            out_v[osl] = ((accs[0] + accs[1]) + (accs[2] + accs[3])
                          + wb_v[osl] + cb_v[osl])
            return carry

        lax.fori_loop(0, CH // L, group, 0)

    pltpu.sync_copy(out_v, out_hbm.at[pl.ds(base, BPW)])


def kernel(word_idx, context_idx, word_embeddings, context_embeddings,
           word_biases, context_biases):
    widx = word_idx.astype(jnp.int32).reshape(NW, NCHUNK, CH)
    cidx = context_idx.astype(jnp.int32).reshape(NW, NCHUNK, CH)
    wtab2, ctab2 = _repack(word_embeddings.T, word_embeddings.T,
                           context_embeddings.T, context_embeddings.T)
    wb = word_biases.reshape(VOCAB)
    cb = context_biases.reshape(VOCAB)
    return _sc_kernel(widx, cidx, wtab2, ctab2, wb, cb)


# speedup vs baseline: 1.0485x; 1.0485x over previous
import functools

import jax
import jax.numpy as jnp
from jax import lax
from jax.experimental import pallas as pl
from jax.experimental.pallas import tpu as pltpu
from jax.experimental.pallas import tpu_sc as plsc

VOCAB = 1000000
EMBED = 64
BATCH = 16384
PAIRW = 2 * EMBED

VBLK = 8192
VBITS = 13
NVB = (VOCAB + VBLK - 1) // VBLK
NPB = (NVB + 1) // 2
PROWS = NPB * VBLK

NC = 2
NS = 16
L = 16
NW = NC * NS
BPW = BATCH // NW
NCHUNK = 4
CH = BPW // NCHUNK


NSUB = 8
SUB = VBLK // NSUB


def _repack_body(wlo_ref, whi_ref, clo_ref, chi_ref, wout_ref, cout_ref):
    for q in range(NSUB):
        rs = slice(q * SUB, (q + 1) * SUB)
        wout_ref[rs, 0:EMBED] = wlo_ref[:, rs].T
        wout_ref[rs, EMBED:PAIRW] = whi_ref[:, rs].T
        cout_ref[rs, 0:EMBED] = clo_ref[:, rs].T
        cout_ref[rs, EMBED:PAIRW] = chi_ref[:, rs].T


def _hi_block(i):
    return (0, jnp.minimum(2 * i + 1, NVB - 1))


_repack = pl.pallas_call(
    _repack_body,
    grid=(NPB,),
    in_specs=[
        pl.BlockSpec((EMBED, VBLK), lambda i: (0, 2 * i)),
        pl.BlockSpec((EMBED, VBLK), _hi_block),
        pl.BlockSpec((EMBED, VBLK), lambda i: (0, 2 * i)),
        pl.BlockSpec((EMBED, VBLK), _hi_block),
    ],
    out_specs=[
        pl.BlockSpec((VBLK, PAIRW), lambda i: (i, 0)),
        pl.BlockSpec((VBLK, PAIRW), lambda i: (i, 0)),
    ],
    out_shape=[
        jax.ShapeDtypeStruct((PROWS, PAIRW), jnp.float32),
        jax.ShapeDtypeStruct((PROWS, PAIRW), jnp.float32),
    ],
)


_mesh = plsc.VectorSubcoreMesh(core_axis_name="c", subcore_axis_name="s")


@functools.partial(
    pl.kernel,
    out_type=jax.ShapeDtypeStruct((BATCH,), jnp.float32),
    mesh=_mesh,
    compiler_params=pltpu.CompilerParams(needs_layout_passes=False,
                                         use_tc_tiling_on_sc=True),
    scratch_types=[
        pltpu.VMEM((NCHUNK, CH), jnp.int32),
        pltpu.VMEM((NCHUNK, CH), jnp.int32),
        pltpu.VMEM((NCHUNK, CH), jnp.int32),
        pltpu.VMEM((NCHUNK, CH), jnp.int32),
        pltpu.VMEM((NCHUNK, CH), jnp.int32),
        pltpu.VMEM((NCHUNK, CH), jnp.int32),
        pltpu.VMEM((CH, PAIRW), jnp.float32),
        pltpu.VMEM((CH, PAIRW), jnp.float32),
        pltpu.VMEM((CH, PAIRW), jnp.float32),
        pltpu.VMEM((CH, PAIRW), jnp.float32),
        pltpu.VMEM((BPW,), jnp.float32),
        pltpu.VMEM((BPW,), jnp.float32),
        pltpu.VMEM((BPW,), jnp.float32),
        pltpu.SemaphoreType.DMA,
        pltpu.SemaphoreType.DMA,
    ],
)
def _sc_kernel(widx_hbm, cidx_hbm, wtab_hbm, ctab_hbm, wb_hbm, cb_hbm,
               out_hbm, widx_v, cidx_v, wp_v, cp_v, wo_v, co_v,
               wrows0, wrows1, crows0, crows1, wb_v, cb_v, out_v, sem, bsem):
    wid = lax.axis_index("s") * NC + lax.axis_index("c")
    base = wid * BPW
    wbufs = (wrows0, wrows1)
    cbufs = (crows0, crows1)

    pltpu.sync_copy(widx_hbm.at[wid], widx_v)
    pltpu.sync_copy(cidx_hbm.at[wid], cidx_v)

    for j in range(NCHUNK):
        for t in range(CH // L):
            sl = pl.ds(t * L, L)
            for iv, pv, ov in ((widx_v, wp_v, wo_v), (cidx_v, cp_v, co_v)):
                v = iv[j, sl]
                k = lax.shift_right_logical(v, VBITS)
                pv[j, sl] = lax.shift_left(lax.shift_right_logical(k, 1), VBITS) + (v & (VBLK - 1))
                ov[j, sl] = (k & 1) * EMBED

    def fire(j):
        return (pltpu.async_copy(wtab_hbm.at[wp_v.at[j]], wbufs[j % 2], sem),
                pltpu.async_copy(ctab_hbm.at[cp_v.at[j]], cbufs[j % 2], sem))

    inflight = fire(0)

    bias_copies = []
    for j in range(NCHUNK):
        sl = pl.ds(j * CH, CH)
        bias_copies.append(pltpu.async_copy(wb_hbm.at[widx_v.at[j]], wb_v.at[sl], bsem))
        bias_copies.append(pltpu.async_copy(cb_hbm.at[cidx_v.at[j]], cb_v.at[sl], bsem))

    lane = lax.iota(jnp.int32, L)

    for j in range(NCHUNK):
        for c in inflight:
            c.wait()
        if j + 1 < NCHUNK:
            inflight = fire(j + 1)
        if j == 0:
            for c in bias_copies:
                c.wait()
        wrows, crows = wbufs[j % 2], cbufs[j % 2]

        def group(g, carry, j=j, wrows=wrows, crows=crows):
            b16 = g * L + lane
            osl = pl.ds(j * CH + g * L, L)
            woff = wo_v[j, pl.ds(g * L, L)]
            coff = co_v[j, pl.ds(g * L, L)]
            acc = jnp.zeros((L,), jnp.float32)
            for d in range(EMBED):
                wv = plsc.load_gather(wrows, [b16, woff + d])
                cv = plsc.load_gather(crows, [b16, coff + d])
                acc = acc + wv * cv
            out_v[osl] = acc + wb_v[osl] + cb_v[osl]
            return carry

        lax.fori_loop(0, CH // L, group, 0)

    pltpu.sync_copy(out_v, out_hbm.at[pl.ds(base, BPW)])


def kernel(word_idx, context_idx, word_embeddings, context_embeddings,
           word_biases, context_biases):
    widx = word_idx.astype(jnp.int32).reshape(NW, NCHUNK, CH)
    cidx = context_idx.astype(jnp.int32).reshape(NW, NCHUNK, CH)
    wtab2, ctab2 = _repack(word_embeddings.T, word_embeddings.T,
                           context_embeddings.T, context_embeddings.T)
    wb = word_biases.reshape(VOCAB)
    cb = context_biases.reshape(VOCAB)
    return _sc_kernel(widx, cidx, wtab2, ctab2, wb, cb)


# speedup vs baseline: 1.0520x; 1.0033x over previous
import functools

import jax
import jax.numpy as jnp
from jax import lax
from jax.experimental import pallas as pl
from jax.experimental.pallas import tpu as pltpu
from jax.experimental.pallas import tpu_sc as plsc

VOCAB = 1000000
EMBED = 64
BATCH = 16384
PAIRW = 2 * EMBED

VBLK = 8192
VBITS = 13
NVB = (VOCAB + VBLK - 1) // VBLK
NPB = (NVB + 1) // 2
PROWS = NPB * VBLK

NC = 2
NS = 16
L = 16
NW = NC * NS
BPW = BATCH // NW
NCHUNK = 4
CH = BPW // NCHUNK


NSUB = 8
SUB = VBLK // NSUB


def _repack_body(w_ref, c_ref, wout_ref, cout_ref):
    for q in range(NSUB):
        rs = slice(q * SUB, (q + 1) * SUB)
        hs = slice(VBLK + q * SUB, VBLK + (q + 1) * SUB)
        wout_ref[rs, 0:EMBED] = w_ref[:, rs].T
        wout_ref[rs, EMBED:PAIRW] = w_ref[:, hs].T
        cout_ref[rs, 0:EMBED] = c_ref[:, rs].T
        cout_ref[rs, EMBED:PAIRW] = c_ref[:, hs].T


_repack = pl.pallas_call(
    _repack_body,
    grid=(NPB,),
    in_specs=[
        pl.BlockSpec((EMBED, 2 * VBLK), lambda i: (0, i)),
        pl.BlockSpec((EMBED, 2 * VBLK), lambda i: (0, i)),
    ],
    out_specs=[
        pl.BlockSpec((VBLK, PAIRW), lambda i: (i, 0)),
        pl.BlockSpec((VBLK, PAIRW), lambda i: (i, 0)),
    ],
    out_shape=[
        jax.ShapeDtypeStruct((PROWS, PAIRW), jnp.float32),
        jax.ShapeDtypeStruct((PROWS, PAIRW), jnp.float32),
    ],
)


_mesh = plsc.VectorSubcoreMesh(core_axis_name="c", subcore_axis_name="s")


@functools.partial(
    pl.kernel,
    out_type=jax.ShapeDtypeStruct((BATCH,), jnp.float32),
    mesh=_mesh,
    compiler_params=pltpu.CompilerParams(needs_layout_passes=False,
                                         use_tc_tiling_on_sc=True),
    scratch_types=[
        pltpu.VMEM((NCHUNK, CH), jnp.int32),
        pltpu.VMEM((NCHUNK, CH), jnp.int32),
        pltpu.VMEM((NCHUNK, CH), jnp.int32),
        pltpu.VMEM((NCHUNK, CH), jnp.int32),
        pltpu.VMEM((NCHUNK, CH), jnp.int32),
        pltpu.VMEM((NCHUNK, CH), jnp.int32),
        pltpu.VMEM((CH, PAIRW), jnp.float32),
        pltpu.VMEM((CH, PAIRW), jnp.float32),
        pltpu.VMEM((CH, PAIRW), jnp.float32),
        pltpu.VMEM((CH, PAIRW), jnp.float32),
        pltpu.VMEM((BPW,), jnp.float32),
        pltpu.VMEM((BPW,), jnp.float32),
        pltpu.VMEM((BPW,), jnp.float32),
        pltpu.SemaphoreType.DMA,
        pltpu.SemaphoreType.DMA,
    ],
)
def _sc_kernel(widx_hbm, cidx_hbm, wtab_hbm, ctab_hbm, wb_hbm, cb_hbm,
               out_hbm, widx_v, cidx_v, wp_v, cp_v, wo_v, co_v,
               wrows0, wrows1, crows0, crows1, wb_v, cb_v, out_v, sem, bsem):
    wid = lax.axis_index("s") * NC + lax.axis_index("c")
    base = wid * BPW
    wbufs = (wrows0, wrows1)
    cbufs = (crows0, crows1)

    pltpu.sync_copy(widx_hbm.at[wid], widx_v)
    pltpu.sync_copy(cidx_hbm.at[wid], cidx_v)

    for j in range(NCHUNK):
        for t in range(CH // L):
            sl = pl.ds(t * L, L)
            for iv, pv, ov in ((widx_v, wp_v, wo_v), (cidx_v, cp_v, co_v)):
                v = iv[j, sl]
                k = lax.shift_right_logical(v, VBITS)
                pv[j, sl] = lax.shift_left(lax.shift_right_logical(k, 1), VBITS) + (v & (VBLK - 1))
                ov[j, sl] = (k & 1) * EMBED

    def fire(j):
        return (pltpu.async_copy(wtab_hbm.at[wp_v.at[j]], wbufs[j % 2], sem),
                pltpu.async_copy(ctab_hbm.at[cp_v.at[j]], cbufs[j % 2], sem))

    inflight = fire(0)

    bias_copies = []
    for j in range(NCHUNK):
        sl = pl.ds(j * CH, CH)
        bias_copies.append(pltpu.async_copy(wb_hbm.at[widx_v.at[j]], wb_v.at[sl], bsem))
        bias_copies.append(pltpu.async_copy(cb_hbm.at[cidx_v.at[j]], cb_v.at[sl], bsem))

    lane = lax.iota(jnp.int32, L)

    for j in range(NCHUNK):
        for c in inflight:
            c.wait()
        if j + 1 < NCHUNK:
            inflight = fire(j + 1)
        if j == 0:
            for c in bias_copies:
                c.wait()
        wrows, crows = wbufs[j % 2], cbufs[j % 2]

        def group(g, carry, j=j, wrows=wrows, crows=crows):
            b16 = g * L + lane
            osl = pl.ds(j * CH + g * L, L)
            woff = wo_v[j, pl.ds(g * L, L)]
            coff = co_v[j, pl.ds(g * L, L)]
            acc = jnp.zeros((L,), jnp.float32)
            for d in range(EMBED):
                wv = plsc.load_gather(wrows, [b16, woff + d])
                cv = plsc.load_gather(crows, [b16, coff + d])
                acc = acc + wv * cv
            out_v[osl] = acc + wb_v[osl] + cb_v[osl]
            return carry

        lax.fori_loop(0, CH // L, group, 0)

    pltpu.sync_copy(out_v, out_hbm.at[pl.ds(base, BPW)])


def kernel(word_idx, context_idx, word_embeddings, context_embeddings,
           word_biases, context_biases):
    widx = word_idx.astype(jnp.int32).reshape(NW, NCHUNK, CH)
    cidx = context_idx.astype(jnp.int32).reshape(NW, NCHUNK, CH)
    wtab2, ctab2 = _repack(word_embeddings.T, context_embeddings.T)
    wb = word_biases[:, 0]
    cb = context_biases[:, 0]
    return _sc_kernel(widx, cidx, wtab2, ctab2, wb, cb)


# speedup vs baseline: 1.0717x; 1.0187x over previous
import functools

import jax
import jax.numpy as jnp
from jax import lax
from jax.experimental import pallas as pl
from jax.experimental.pallas import tpu as pltpu
from jax.experimental.pallas import tpu_sc as plsc

VOCAB = 1000000
EMBED = 64
BATCH = 16384
PAIRW = 2 * EMBED

VBLK = 8192
VBITS = 13
NVB = (VOCAB + VBLK - 1) // VBLK
NPB = (NVB + 1) // 2
PROWS = NPB * VBLK

NC = 2
NS = 16
L = 16
NW = NC * NS
BPW = BATCH // NW
NCHUNK = 4
CH = BPW // NCHUNK


NSUB = 8
SUB = VBLK // NSUB


def _repack_body(w_ref, c_ref, wout_ref, cout_ref):
    for q in range(NSUB):
        rs = slice(q * SUB, (q + 1) * SUB)
        hs = slice(VBLK + q * SUB, VBLK + (q + 1) * SUB)
        wout_ref[rs, 0:EMBED] = w_ref[:, rs].T
        wout_ref[rs, EMBED:PAIRW] = w_ref[:, hs].T
        cout_ref[rs, 0:EMBED] = c_ref[:, rs].T
        cout_ref[rs, EMBED:PAIRW] = c_ref[:, hs].T


_repack = pl.pallas_call(
    _repack_body,
    grid=(NPB,),
    in_specs=[
        pl.BlockSpec((EMBED, 2 * VBLK), lambda i: (0, i)),
        pl.BlockSpec((EMBED, 2 * VBLK), lambda i: (0, i)),
    ],
    out_specs=[
        pl.BlockSpec((VBLK, PAIRW), lambda i: (i, 0)),
        pl.BlockSpec((VBLK, PAIRW), lambda i: (i, 0)),
    ],
    out_shape=[
        jax.ShapeDtypeStruct((PROWS, PAIRW), jnp.float32),
        jax.ShapeDtypeStruct((PROWS, PAIRW), jnp.float32),
    ],
)


_mesh = plsc.VectorSubcoreMesh(core_axis_name="c", subcore_axis_name="s")


@functools.partial(
    pl.kernel,
    out_type=jax.ShapeDtypeStruct((BATCH,), jnp.float32),
    mesh=_mesh,
    compiler_params=pltpu.CompilerParams(needs_layout_passes=False,
                                         use_tc_tiling_on_sc=True),
    scratch_types=[
        pltpu.VMEM((NCHUNK, CH), jnp.int32),
        pltpu.VMEM((NCHUNK, CH), jnp.int32),
        pltpu.VMEM((NCHUNK, CH), jnp.int32),
        pltpu.VMEM((NCHUNK, CH), jnp.int32),
        pltpu.VMEM((NCHUNK, CH), jnp.int32),
        pltpu.VMEM((NCHUNK, CH), jnp.int32),
        pltpu.VMEM((CH, PAIRW), jnp.float32),
        pltpu.VMEM((CH, PAIRW), jnp.float32),
        pltpu.VMEM((CH, PAIRW), jnp.float32),
        pltpu.VMEM((CH, PAIRW), jnp.float32),
        pltpu.VMEM((BPW,), jnp.float32),
        pltpu.VMEM((BPW,), jnp.float32),
        pltpu.VMEM((BPW,), jnp.float32),
        pltpu.SemaphoreType.DMA,
        pltpu.SemaphoreType.DMA,
    ],
)
def _sc_kernel(widx_hbm, cidx_hbm, wtab_hbm, ctab_hbm, wb_hbm, cb_hbm,
               out_hbm, widx_v, cidx_v, wp_v, cp_v, wo_v, co_v,
               wrows0, wrows1, crows0, crows1, wb_v, cb_v, out_v, sem, bsem):
    wid = lax.axis_index("s") * NC + lax.axis_index("c")
    base = wid * BPW
    wbufs = (wrows0, wrows1)
    cbufs = (crows0, crows1)

    pltpu.sync_copy(widx_hbm.at[wid], widx_v)
    pltpu.sync_copy(cidx_hbm.at[wid], cidx_v)

    for j in range(NCHUNK):
        for t in range(CH // L):
            sl = pl.ds(t * L, L)
            for iv, pv, ov in ((widx_v, wp_v, wo_v), (cidx_v, cp_v, co_v)):
                v = iv[j, sl]
                k = lax.shift_right_logical(v, VBITS)
                pv[j, sl] = lax.shift_left(lax.shift_right_logical(k, 1), VBITS) + (v & (VBLK - 1))
                ov[j, sl] = (k & 1) * EMBED

    def fire(j):
        return (pltpu.async_copy(wtab_hbm.at[wp_v.at[j]], wbufs[j % 2], sem),
                pltpu.async_copy(ctab_hbm.at[cp_v.at[j]], cbufs[j % 2], sem))

    inflight = fire(0)

    bias_copies = []
    for j in range(NCHUNK):
        sl = pl.ds(j * CH, CH)
        bias_copies.append(pltpu.async_copy(wb_hbm.at[widx_v.at[j]], wb_v.at[sl], bsem))
        bias_copies.append(pltpu.async_copy(cb_hbm.at[cidx_v.at[j]], cb_v.at[sl], bsem))

    lane = lax.iota(jnp.int32, L)

    for j in range(NCHUNK):
        for c in inflight:
            c.wait()
        if j + 1 < NCHUNK:
            inflight = fire(j + 1)
        if j == 0:
            for c in bias_copies:
                c.wait()
        wrows, crows = wbufs[j % 2], cbufs[j % 2]

        def group(g, carry, j=j, wrows=wrows, crows=crows):
            b16 = g * L + lane
            osl = pl.ds(j * CH + g * L, L)
            woff = wo_v[j, pl.ds(g * L, L)]
            coff = co_v[j, pl.ds(g * L, L)]
            acc = jnp.zeros((L,), jnp.float32)
            for d in range(EMBED):
                wv = plsc.load_gather(wrows, [b16, woff + d])
                cv = plsc.load_gather(crows, [b16, coff + d])
                acc = acc + wv * cv
            out_v[osl] = acc + wb_v[osl] + cb_v[osl]
            return carry

        lax.fori_loop(0, CH // L, group, 0, unroll=2)

    pltpu.sync_copy(out_v, out_hbm.at[pl.ds(base, BPW)])


def kernel(word_idx, context_idx, word_embeddings, context_embeddings,
           word_biases, context_biases):
    widx = word_idx.astype(jnp.int32).reshape(NW, NCHUNK, CH)
    cidx = context_idx.astype(jnp.int32).reshape(NW, NCHUNK, CH)
    wtab2, ctab2 = _repack(word_embeddings.T, context_embeddings.T)
    wb = word_biases[:, 0]
    cb = context_biases[:, 0]
    return _sc_kernel(widx, cidx, wtab2, ctab2, wb, cb)
